# Initial kernel scaffold; baseline (speedup 1.0000x reference)
import jax
import jax.numpy as jnp
from jax.experimental import pallas as pl


def kernel(x, edge_index, batch, eigens, embed, W, a_src, a_dst, a_eig, bias, W1, b1, W2, b2):
    raise NotImplementedError("write your pallas kernel here")


# speedup vs baseline: 1.3822x; 1.3822x over previous
import functools

import jax
import jax.numpy as jnp
from jax import lax
from jax.experimental import pallas as pl
from jax.experimental.pallas import tpu as pltpu
from jax.experimental.pallas import tpu_sc as plsc

N = 50000
E = 800000
HID = 64
L = 3
EIG = 8
NG = 64

NPAD = 50176
BLK = 1024
GRID = NPAD // BLK


def _make_sc_gather(V, D, B):
    info = plsc.get_sparse_core_info()
    NC, NS = info.num_cores, info.num_subcores
    NW = NC * NS
    assert B % (16 * NW) == 0
    b_per_w = B // NW
    chunk = b_per_w // 2
    mesh = plsc.VectorSubcoreMesh(core_axis_name="c", subcore_axis_name="s")

    @functools.partial(
        pl.kernel, mesh=mesh,
        out_type=jax.ShapeDtypeStruct((B, D), jnp.float32),
        scratch_types=[
            pltpu.VMEM((chunk,), jnp.int32),
            pltpu.VMEM((chunk, D), jnp.float32),
            pltpu.SemaphoreType.DMA,
        ],
    )
    def k(table_hbm, idx_hbm, out_hbm, idx_v, rows_v, sem):
        wid = lax.axis_index("s") * NC + lax.axis_index("c")
        for t in range(2):
            base = wid * b_per_w + t * chunk
            pltpu.sync_copy(idx_hbm.at[pl.ds(base, chunk)], idx_v)
            pltpu.async_copy(table_hbm.at[idx_v], rows_v, sem).wait()
            pltpu.sync_copy(rows_v, out_hbm.at[pl.ds(base, chunk)])

    return k


def _dense_layer_kernel(h_ref, eig_ref, w_ref, a_ref, ae_ref, h1_ref, uv_ref):
    h1 = jnp.dot(h_ref[...], w_ref[...], preferred_element_type=jnp.float32)
    h1_ref[...] = h1
    uv_ref[...] = (
        jnp.dot(h1, a_ref[...], preferred_element_type=jnp.float32)
        + jnp.dot(eig_ref[...], ae_ref[...], preferred_element_type=jnp.float32)
    )


def _dense_layer(h_pad, eig_pad, W_i, A, Ae):
    return pl.pallas_call(
        _dense_layer_kernel,
        grid=(GRID,),
        in_specs=[
            pl.BlockSpec((BLK, 128), lambda i: (i, 0)),
            pl.BlockSpec((BLK, 128), lambda i: (i, 0)),
            pl.BlockSpec((128, HID), lambda i: (0, 0)),
            pl.BlockSpec((HID, 128), lambda i: (0, 0)),
            pl.BlockSpec((128, 128), lambda i: (0, 0)),
        ],
        out_specs=[
            pl.BlockSpec((BLK, HID), lambda i: (i, 0)),
            pl.BlockSpec((BLK, 128), lambda i: (i, 0)),
        ],
        out_shape=[
            jax.ShapeDtypeStruct((NPAD, HID), jnp.float32),
            jax.ShapeDtypeStruct((NPAD, 128), jnp.float32),
        ],
    )(h_pad, eig_pad, W_i, A, Ae)


def _pool_mlp_kernel(h_ref, b_ref, w1_ref, b1_ref, w2_ref, b2_ref,
                     y_ref, acc_ref):
    i = pl.program_id(0)

    @pl.when(i == 0)
    def _():
        acc_ref[...] = jnp.zeros_like(acc_ref)

    gids = lax.broadcasted_iota(jnp.int32, (BLK, NG), 1)
    onehot = jnp.where(b_ref[...] == gids, 1.0, 0.0)
    acc_ref[...] += lax.dot_general(
        onehot, h_ref[...], (((0,), (0,)), ((), ())),
        preferred_element_type=jnp.float32)

    @pl.when(i == GRID - 1)
    def _():
        z = jnp.maximum(
            jnp.dot(acc_ref[...], w1_ref[...],
                    preferred_element_type=jnp.float32) + b1_ref[...], 0.0)
        y_ref[...] = jnp.dot(z, w2_ref[...],
                             preferred_element_type=jnp.float32) + b2_ref[...]


def _pool_mlp(h_pad, batch_pad, W1, b1, W2p, b2p):
    return pl.pallas_call(
        _pool_mlp_kernel,
        grid=(GRID,),
        in_specs=[
            pl.BlockSpec((BLK, 128), lambda i: (i, 0)),
            pl.BlockSpec((BLK, 1), lambda i: (i, 0)),
            pl.BlockSpec((128, HID), lambda i: (0, 0)),
            pl.BlockSpec((1, HID), lambda i: (0, 0)),
            pl.BlockSpec((HID, 128), lambda i: (0, 0)),
            pl.BlockSpec((1, 128), lambda i: (0, 0)),
        ],
        out_specs=pl.BlockSpec((NG, 128), lambda i: (0, 0)),
        out_shape=jax.ShapeDtypeStruct((NG, 128), jnp.float32),
        scratch_shapes=[pltpu.VMEM((NG, 128), jnp.float32)],
    )(h_pad, batch_pad, W1, b1, W2p, b2p)


def kernel(x, edge_index, batch, eigens, embed, W, a_src, a_dst, a_eig,
           bias, W1, b1, W2, b2):
    src = edge_index[0].astype(jnp.int32)
    dst = edge_index[1].astype(jnp.int32)

    idx = jnp.zeros((NPAD,), jnp.int32).at[:N].set(x[:, 0].astype(jnp.int32))
    embed_p = jnp.zeros((embed.shape[0], 128), jnp.float32).at[:, :HID].set(embed)
    h_pad = _make_sc_gather(embed.shape[0], 128, NPAD)(embed_p, idx)

    eig_pad = jnp.zeros((NPAD, 128), jnp.float32).at[:N, :EIG].set(eigens)

    for i in range(L):
        A = jnp.zeros((HID, 128), jnp.float32)
        A = A.at[:, 0].set(a_src[i]).at[:, 1].set(a_dst[i])
        Ae = jnp.zeros((128, 128), jnp.float32)
        Ae = Ae.at[:EIG, 0].set(a_eig[i]).at[:EIG, 1].set(a_eig[i])

        Wp = jnp.zeros((128, HID), jnp.float32).at[:HID].set(W[i])
        h1_pad, uv = _dense_layer(h_pad, eig_pad, Wp, A, Ae)
        h1 = h1_pad[:N]
        u = uv[:N, 0]
        v = uv[:N, 1]

        logits = jax.nn.leaky_relu(u[src] + v[dst], 0.2)
        m = jax.ops.segment_max(logits, dst, num_segments=N)
        m = jnp.where(jnp.isfinite(m), m, 0.0)
        ex = jnp.exp(logits - m[dst])
        denom = jax.ops.segment_sum(ex, dst, num_segments=N)
        attn = ex / (denom[dst] + 1e-16)
        out = jax.ops.segment_sum(h1[src] * attn[:, None], dst, num_segments=N)
        h = jnp.maximum(out + bias[i], 0.0)
        h_pad = jnp.zeros((NPAD, 128), jnp.float32).at[:N, :HID].set(h)

    batch_pad = jnp.full((NPAD, 1), -1, jnp.int32).at[:N, 0].set(
        batch.astype(jnp.int32))
    W1p = jnp.zeros((128, HID), jnp.float32).at[:HID].set(W1)
    W2p = jnp.zeros((HID, 128), jnp.float32).at[:, :1].set(W2)
    b2p = jnp.zeros((1, 128), jnp.float32).at[0, 0].set(b2[0])
    b1r = b1.reshape(1, HID)

    y = _pool_mlp(h_pad, batch_pad, W1p, b1r, W2p, b2p)
    return y[:, 0]


# speedup vs baseline: 1.8307x; 1.3244x over previous
import functools

import jax
import jax.numpy as jnp
from jax import lax
from jax.experimental import pallas as pl
from jax.experimental.pallas import tpu as pltpu
from jax.experimental.pallas import tpu_sc as plsc

N = 50000
E = 800000
HID = 64
L = 3
EIG = 8
NG = 64

NPAD = 50176
BLK = 1024
GRID = NPAD // BLK


def _make_sc_gather(V, D, B):
    info = plsc.get_sparse_core_info()
    NC, NS = info.num_cores, info.num_subcores
    NW = NC * NS
    assert B % (16 * NW) == 0
    b_per_w = B // NW
    chunk = b_per_w // 2
    mesh = plsc.VectorSubcoreMesh(core_axis_name="c", subcore_axis_name="s")

    @functools.partial(
        pl.kernel, mesh=mesh,
        out_type=jax.ShapeDtypeStruct((B, D), jnp.float32),
        scratch_types=[
            pltpu.VMEM((chunk,), jnp.int32),
            pltpu.VMEM((chunk, D), jnp.float32),
            pltpu.SemaphoreType.DMA,
        ],
    )
    def k(table_hbm, idx_hbm, out_hbm, idx_v, rows_v, sem):
        wid = lax.axis_index("s") * NC + lax.axis_index("c")
        for t in range(2):
            base = wid * b_per_w + t * chunk
            pltpu.sync_copy(idx_hbm.at[pl.ds(base, chunk)], idx_v)
            pltpu.async_copy(table_hbm.at[idx_v], rows_v, sem).wait()
            pltpu.sync_copy(rows_v, out_hbm.at[pl.ds(base, chunk)])

    return k


def _dense_layer_kernel(h_ref, eig_ref, w_ref, a_ref, ae_ref, h1_ref, uv_ref):
    h1 = jnp.dot(h_ref[...], w_ref[...], preferred_element_type=jnp.float32)
    h1_ref[...] = h1
    uv_ref[...] = (
        jnp.dot(h1, a_ref[...], preferred_element_type=jnp.float32)
        + jnp.dot(eig_ref[...], ae_ref[...], preferred_element_type=jnp.float32)
    )


def _dense_layer(h_pad, eig_pad, W_i, A, Ae):
    return pl.pallas_call(
        _dense_layer_kernel,
        grid=(GRID,),
        in_specs=[
            pl.BlockSpec((BLK, 128), lambda i: (i, 0)),
            pl.BlockSpec((BLK, 128), lambda i: (i, 0)),
            pl.BlockSpec((128, HID), lambda i: (0, 0)),
            pl.BlockSpec((HID, 128), lambda i: (0, 0)),
            pl.BlockSpec((128, 128), lambda i: (0, 0)),
        ],
        out_specs=[
            pl.BlockSpec((BLK, HID), lambda i: (i, 0)),
            pl.BlockSpec((BLK, 128), lambda i: (i, 0)),
        ],
        out_shape=[
            jax.ShapeDtypeStruct((NPAD, HID), jnp.float32),
            jax.ShapeDtypeStruct((NPAD, 128), jnp.float32),
        ],
    )(h_pad, eig_pad, W_i, A, Ae)


def _pool_mlp_kernel(h_ref, b_ref, w1_ref, b1_ref, w2_ref, b2_ref,
                     y_ref, acc_ref):
    i = pl.program_id(0)

    @pl.when(i == 0)
    def _():
        acc_ref[...] = jnp.zeros_like(acc_ref)

    gids = lax.broadcasted_iota(jnp.int32, (BLK, NG), 1)
    onehot = jnp.where(b_ref[...] == gids, 1.0, 0.0)
    acc_ref[...] += lax.dot_general(
        onehot, h_ref[...], (((0,), (0,)), ((), ())),
        preferred_element_type=jnp.float32)

    @pl.when(i == GRID - 1)
    def _():
        z = jnp.maximum(
            jnp.dot(acc_ref[...], w1_ref[...],
                    preferred_element_type=jnp.float32) + b1_ref[...], 0.0)
        y_ref[...] = jnp.dot(z, w2_ref[...],
                             preferred_element_type=jnp.float32) + b2_ref[...]


def _pool_mlp(h_pad, batch_pad, W1, b1, W2p, b2p):
    return pl.pallas_call(
        _pool_mlp_kernel,
        grid=(GRID,),
        in_specs=[
            pl.BlockSpec((BLK, 128), lambda i: (i, 0)),
            pl.BlockSpec((BLK, 1), lambda i: (i, 0)),
            pl.BlockSpec((128, HID), lambda i: (0, 0)),
            pl.BlockSpec((1, HID), lambda i: (0, 0)),
            pl.BlockSpec((HID, 128), lambda i: (0, 0)),
            pl.BlockSpec((1, 128), lambda i: (0, 0)),
        ],
        out_specs=pl.BlockSpec((NG, 128), lambda i: (0, 0)),
        out_shape=jax.ShapeDtypeStruct((NG, 128), jnp.float32),
        scratch_shapes=[pltpu.VMEM((NG, 128), jnp.float32)],
    )(h_pad, batch_pad, W1, b1, W2p, b2p)


def kernel(x, edge_index, batch, eigens, embed, W, a_src, a_dst, a_eig,
           bias, W1, b1, W2, b2):
    src = edge_index[0].astype(jnp.int32)
    dst = edge_index[1].astype(jnp.int32)

    idx = jnp.zeros((NPAD,), jnp.int32).at[:N].set(x[:, 0].astype(jnp.int32))
    embed_p = jnp.zeros((embed.shape[0], 128), jnp.float32).at[:, :HID].set(embed)
    h_pad = _make_sc_gather(embed.shape[0], 128, NPAD)(embed_p, idx)

    eig_pad = jnp.zeros((NPAD, 128), jnp.float32).at[:N, :EIG].set(eigens)

    for i in range(L):
        A = jnp.zeros((HID, 128), jnp.float32)
        A = A.at[:, 0].set(a_src[i]).at[:, 1].set(a_dst[i])
        Ae = jnp.zeros((128, 128), jnp.float32)
        Ae = Ae.at[:EIG, 0].set(a_eig[i]).at[:EIG, 1].set(a_eig[i])

        Wp = jnp.zeros((128, HID), jnp.float32).at[:HID].set(W[i])
        h1_pad, uv = _dense_layer(h_pad, eig_pad, Wp, A, Ae)
        h1 = h1_pad[:N]
        u = uv[:N, 0]
        v = uv[:N, 1]

        logits = jax.nn.leaky_relu(u[src] + v[dst], 0.2)
        ex = jnp.exp(logits)
        denom = jax.ops.segment_sum(ex, dst, num_segments=N)
        attn = ex / (denom[dst] + 1e-16)
        out = jax.ops.segment_sum(h1[src] * attn[:, None], dst, num_segments=N)
        h = jnp.maximum(out + bias[i], 0.0)
        h_pad = jnp.zeros((NPAD, 128), jnp.float32).at[:N, :HID].set(h)

    batch_pad = jnp.full((NPAD, 1), -1, jnp.int32).at[:N, 0].set(
        batch.astype(jnp.int32))
    W1p = jnp.zeros((128, HID), jnp.float32).at[:HID].set(W1)
    W2p = jnp.zeros((HID, 128), jnp.float32).at[:, :1].set(W2)
    b2p = jnp.zeros((1, 128), jnp.float32).at[0, 0].set(b2[0])
    b1r = b1.reshape(1, HID)

    y = _pool_mlp(h_pad, batch_pad, W1p, b1r, W2p, b2p)
    return y[:, 0]
